# baseline (speedup 1.0000x reference)
import functools

import jax
import jax.numpy as jnp
from jax import lax
from jax.experimental import pallas as pl
from jax.experimental.pallas import tpu as pltpu
from jax.experimental.pallas import tpu_sc as plsc

N_NODES = 10000
N_EDGES = 320000
D_FEAT = 128

NC = 2
NS = 16
NW = NC * NS
EDGES_PER_TILE = N_EDGES // NW
CHUNK = 80
N_CHUNKS = EDGES_PER_TILE // CHUNK
SUPER = 5
SUPER_ROWS = SUPER * CHUNK
N_SUPER = EDGES_PER_TILE // SUPER_ROWS
N_PAD = 10240
ROWS_PER_TILE = N_PAD // NS
DRAIN = 128
N_DRAIN = ROWS_PER_TILE // DRAIN


def _sc_segment_sum(dst_idx, edge_attr):
    mesh = plsc.VectorSubcoreMesh(core_axis_name="c", subcore_axis_name="s")

    @functools.partial(
        pl.kernel,
        mesh=mesh,
        out_type=jax.ShapeDtypeStruct((NC, N_PAD, D_FEAT), jnp.float32),
        scratch_types=[
            pltpu.VMEM((4, CHUNK, D_FEAT), jnp.float32),
            pltpu.VMEM((4, 1, CHUNK), jnp.int32),
            pltpu.VMEM_SHARED((N_PAD, D_FEAT), jnp.float32),
            pltpu.SemaphoreType.DMA,
            pltpu.SemaphoreType.DMA,
            pltpu.SemaphoreType.DMA,
        ],
    )
    def seg_sum(idx_hbm, ea_hbm, out_hbm, buf_v, idx_v, acc_s,
                ld_sem, ix_sem, sc_sem):
        cid = lax.axis_index("c")
        sid = lax.axis_index("s")
        wid = sid * NC + cid
        base = wid * EDGES_PER_TILE

        def load(i, slot):
            pltpu.async_copy(
                ea_hbm.at[pl.ds(base + i * CHUNK, CHUNK)], buf_v.at[slot],
                ld_sem)
            pltpu.async_copy(idx_hbm.at[wid, i], idx_v.at[slot], ix_sem)

        def load_wait(slot):
            pltpu.make_async_copy(
                ea_hbm.at[pl.ds(base, CHUNK)], buf_v.at[slot], ld_sem).wait()
            pltpu.make_async_copy(
                idx_hbm.at[wid, 0], idx_v.at[slot], ix_sem).wait()

        def scatter_desc(slot):
            return pltpu.make_async_copy(
                buf_v.at[slot], acc_s.at[idx_v.at[slot, 0]], sc_sem)

        load(0, 0)
        load(1, 1)
        load(2, 2)

        zval = jnp.zeros((16,), jnp.float32)

        def zrow(i, _):
            for j in range(D_FEAT // 16):
                buf_v[3, i, pl.ds(j * 16, 16)] = zval
            return 0

        lax.fori_loop(0, CHUNK, zrow, 0)
        row0 = sid * ROWS_PER_TILE
        for t in range(ROWS_PER_TILE // CHUNK):
            pltpu.sync_copy(buf_v.at[3],
                            acc_s.at[pl.ds(row0 + t * CHUNK, CHUNK)])
        plsc.subcore_barrier()

        def body(i, _):
            slot = lax.rem(i, 4)
            load_wait(slot)

            @pl.when(i > 0)
            def _():
                scatter_desc(lax.rem(i - 1, 4)).wait()

            scatter_desc(slot).start(add=True)

            @pl.when(i + 3 < N_CHUNKS)
            def _():
                load(i + 3, lax.rem(i + 3, 4))
            return 0

        lax.fori_loop(0, N_CHUNKS, body, 0)
        scatter_desc(lax.rem(N_CHUNKS - 1, 4)).wait()
        plsc.subcore_barrier()

        for t in range(ROWS_PER_TILE // CHUNK):
            r = row0 + t * CHUNK
            pltpu.sync_copy(acc_s.at[pl.ds(r, CHUNK)], buf_v.at[0])
            pltpu.sync_copy(buf_v.at[0], out_hbm.at[cid, pl.ds(r, CHUNK)])

    return seg_sum(dst_idx, edge_attr)


def _mlp_body(x_ref, a0_ref, a1_ref, w1_ref, b1_ref, w2_ref, b2_ref,
              w3_ref, b3_ref, o_ref):
    xb = x_ref[...]
    agg = a0_ref[0] + a1_ref[0]
    h = jnp.dot(xb, w1_ref[0:D_FEAT, :], preferred_element_type=jnp.float32)
    h = h + jnp.dot(agg, w1_ref[D_FEAT:, :], preferred_element_type=jnp.float32)
    h = h + b1_ref[...]
    h = h * jax.nn.sigmoid(h)
    h = jnp.dot(h, w2_ref[...], preferred_element_type=jnp.float32) + b2_ref[...]
    h = h * jax.nn.sigmoid(h)
    o_ref[...] = xb + jnp.dot(h, w3_ref[...], preferred_element_type=jnp.float32) \
        + b3_ref[...]


def _tc_mlp(x, partials, W1, b1, W2, b2, W3, b3):
    BLK = 1000
    grid = (N_NODES // BLK,)
    rowspec = pl.BlockSpec((BLK, D_FEAT), lambda i: (i, 0))
    pspec = lambda c: pl.BlockSpec((1, BLK, D_FEAT), lambda i: (c, i, 0))
    whole = lambda shape: pl.BlockSpec(shape, lambda i: (0,) * len(shape))
    return pl.pallas_call(
        _mlp_body,
        grid=grid,
        in_specs=[
            rowspec, pspec(0), pspec(1),
            whole(W1.shape), whole(b1.shape),
            whole(W2.shape), whole(b2.shape),
            whole(W3.shape), whole(b3.shape),
        ],
        out_specs=rowspec,
        out_shape=jax.ShapeDtypeStruct((N_NODES, D_FEAT), jnp.float32),
    )(x, partials, partials, W1, b1, W2, b2, W3, b3)


def kernel(x, edge_index, edge_attr, W1, b1, W2, b2, W3, b3):
    dst_idx = edge_index[1].reshape(NW, N_CHUNKS, 1, CHUNK)
    partials = _sc_segment_sum(dst_idx, edge_attr)
    return _tc_mlp(x, partials,
                   W1, b1.reshape(1, -1), W2, b2.reshape(1, -1),
                   W3, b3.reshape(1, -1))


# speedup vs baseline: 1.0085x; 1.0085x over previous
import functools

import jax
import jax.numpy as jnp
from jax import lax
from jax.experimental import pallas as pl
from jax.experimental.pallas import tpu as pltpu
from jax.experimental.pallas import tpu_sc as plsc

N_NODES = 10000
N_EDGES = 320000
D_FEAT = 128

NC = 2
NS = 16
NW = NC * NS
EDGES_PER_TILE = N_EDGES // NW
CHUNK = 80
N_CHUNKS = EDGES_PER_TILE // CHUNK
SUPER = 5
SUPER_ROWS = SUPER * CHUNK
N_SUPER = EDGES_PER_TILE // SUPER_ROWS
N_PAD = 10240
ROWS_PER_TILE = N_PAD // NS
DRAIN = 128
N_DRAIN = ROWS_PER_TILE // DRAIN


def _sc_segment_sum(dst_idx, edge_attr):
    mesh = plsc.VectorSubcoreMesh(core_axis_name="c", subcore_axis_name="s")

    @functools.partial(
        pl.kernel,
        mesh=mesh,
        out_type=jax.ShapeDtypeStruct((NC, N_PAD, D_FEAT), jnp.float32),
        scratch_types=[
            pltpu.VMEM((3, CHUNK, D_FEAT), jnp.float32),
            pltpu.VMEM((N_CHUNKS, CHUNK), jnp.int32),
            pltpu.VMEM_SHARED((N_PAD, D_FEAT), jnp.float32),
            pltpu.SemaphoreType.DMA,
            pltpu.SemaphoreType.DMA,
        ],
    )
    def seg_sum(idx_hbm, ea_hbm, out_hbm, buf_v, idx_v, acc_s,
                ld_sem, sc_sem):
        cid = lax.axis_index("c")
        sid = lax.axis_index("s")
        wid = sid * NC + cid
        base = wid * EDGES_PER_TILE

        def load(i, slot):
            return pltpu.async_copy(
                ea_hbm.at[pl.ds(base + i * CHUNK, CHUNK)], buf_v.at[slot],
                ld_sem)

        def scatter_desc(i, slot):
            return pltpu.make_async_copy(
                buf_v.at[slot], acc_s.at[idx_v.at[i]], sc_sem)

        idx_cp = pltpu.async_copy(idx_hbm.at[wid], idx_v, sc_sem)
        load(0, 0)
        load(1, 1)

        zval = jnp.zeros((16,), jnp.float32)

        def zrow(i, _):
            for j in range(D_FEAT // 16):
                buf_v[2, i, pl.ds(j * 16, 16)] = zval
            return 0

        lax.fori_loop(0, CHUNK, zrow, 0)
        row0 = sid * ROWS_PER_TILE
        for t in range(ROWS_PER_TILE // CHUNK):
            pltpu.sync_copy(buf_v.at[2],
                            acc_s.at[pl.ds(row0 + t * CHUNK, CHUNK)])
        idx_cp.wait()
        plsc.subcore_barrier()

        def body(i, _):
            slot = lax.rem(i, 3)
            pltpu.make_async_copy(
                ea_hbm.at[pl.ds(base, CHUNK)], buf_v.at[slot], ld_sem
            ).wait()

            @pl.when(i > 0)
            def _():
                scatter_desc(i - 1, lax.rem(i - 1, 3)).wait()

            scatter_desc(i, slot).start(add=True)

            @pl.when(i + 2 < N_CHUNKS)
            def _():
                load(i + 2, lax.rem(i + 2, 3))
            return 0

        lax.fori_loop(0, N_CHUNKS, body, 0)
        scatter_desc(N_CHUNKS - 1, lax.rem(N_CHUNKS - 1, 3)).wait()
        plsc.subcore_barrier()

        for t in range(ROWS_PER_TILE // CHUNK):
            r = row0 + t * CHUNK
            pltpu.sync_copy(acc_s.at[pl.ds(r, CHUNK)], buf_v.at[0])
            pltpu.sync_copy(buf_v.at[0], out_hbm.at[cid, pl.ds(r, CHUNK)])

    return seg_sum(dst_idx, edge_attr)


def _mlp_body(x_ref, a0_ref, a1_ref, w1_ref, b1_ref, w2_ref, b2_ref,
              w3_ref, b3_ref, o_ref):
    xb = x_ref[...]
    agg = a0_ref[0] + a1_ref[0]
    h = jnp.dot(xb, w1_ref[0:D_FEAT, :], preferred_element_type=jnp.float32)
    h = h + jnp.dot(agg, w1_ref[D_FEAT:, :], preferred_element_type=jnp.float32)
    h = h + b1_ref[...]
    h = h * jax.nn.sigmoid(h)
    h = jnp.dot(h, w2_ref[...], preferred_element_type=jnp.float32) + b2_ref[...]
    h = h * jax.nn.sigmoid(h)
    o_ref[...] = xb + jnp.dot(h, w3_ref[...], preferred_element_type=jnp.float32) \
        + b3_ref[...]


def _tc_mlp(x, partials, W1, b1, W2, b2, W3, b3):
    BLK = 1000
    grid = (N_NODES // BLK,)
    rowspec = pl.BlockSpec((BLK, D_FEAT), lambda i: (i, 0))
    pspec = lambda c: pl.BlockSpec((1, BLK, D_FEAT), lambda i: (c, i, 0))
    whole = lambda shape: pl.BlockSpec(shape, lambda i: (0,) * len(shape))
    return pl.pallas_call(
        _mlp_body,
        grid=grid,
        in_specs=[
            rowspec, pspec(0), pspec(1),
            whole(W1.shape), whole(b1.shape),
            whole(W2.shape), whole(b2.shape),
            whole(W3.shape), whole(b3.shape),
        ],
        out_specs=rowspec,
        out_shape=jax.ShapeDtypeStruct((N_NODES, D_FEAT), jnp.float32),
    )(x, partials, partials, W1, b1, W2, b2, W3, b3)


def kernel(x, edge_index, edge_attr, W1, b1, W2, b2, W3, b3):
    dst_idx = edge_index[1].reshape(NW, N_CHUNKS, CHUNK)
    partials = _sc_segment_sum(dst_idx, edge_attr)
    return _tc_mlp(x, partials,
                   W1, b1.reshape(1, -1), W2, b2.reshape(1, -1),
                   W3, b3.reshape(1, -1))


# speedup vs baseline: 1.0781x; 1.0690x over previous
import functools

import jax
import jax.numpy as jnp
from jax import lax
from jax.experimental import pallas as pl
from jax.experimental.pallas import tpu as pltpu
from jax.experimental.pallas import tpu_sc as plsc

N_NODES = 10000
N_EDGES = 320000
D_FEAT = 128

NC = 2
NS = 16
NW = NC * NS
EDGES_PER_TILE = N_EDGES // NW
CHUNK = 80
N_CHUNKS = EDGES_PER_TILE // CHUNK
SUPER = 5
SUPER_ROWS = SUPER * CHUNK
N_SUPER = EDGES_PER_TILE // SUPER_ROWS
N_PAD = 10240
ROWS_PER_TILE = N_PAD // NS
DRAIN = 128
N_DRAIN = ROWS_PER_TILE // DRAIN


def _sc_segment_sum(dst_idx, edge_attr):
    mesh = plsc.VectorSubcoreMesh(core_axis_name="c", subcore_axis_name="s")

    @functools.partial(
        pl.kernel,
        mesh=mesh,
        out_type=jax.ShapeDtypeStruct((NC, N_PAD, D_FEAT), jnp.float32),
        scratch_types=[
            pltpu.VMEM((3, CHUNK, D_FEAT), jnp.float32),
            pltpu.VMEM((N_CHUNKS, CHUNK), jnp.int32),
            pltpu.VMEM_SHARED((N_PAD, D_FEAT), jnp.float32),
            pltpu.SemaphoreType.DMA,
            pltpu.SemaphoreType.DMA,
        ],
    )
    def seg_sum(idx_hbm, ea_hbm, out_hbm, buf_v, idx_v, acc_s,
                ld_sem, sc_sem):
        cid = lax.axis_index("c")
        sid = lax.axis_index("s")
        wid = sid * NC + cid
        base = wid * EDGES_PER_TILE
        idx_hbm = idx_hbm.at[1]

        def load(i, slot):
            return pltpu.async_copy(
                ea_hbm.at[pl.ds(base + i * CHUNK, CHUNK)], buf_v.at[slot],
                ld_sem)

        def scatter_desc(i, slot):
            return pltpu.make_async_copy(
                buf_v.at[slot], acc_s.at[idx_v.at[i]], sc_sem)

        idx_cp = pltpu.async_copy(idx_hbm.at[wid], idx_v, sc_sem)
        load(0, 0)
        load(1, 1)

        zval = jnp.zeros((16,), jnp.float32)

        def zrow(i, _):
            for j in range(D_FEAT // 16):
                buf_v[2, i, pl.ds(j * 16, 16)] = zval
            return 0

        lax.fori_loop(0, CHUNK, zrow, 0)
        row0 = sid * ROWS_PER_TILE
        for t in range(ROWS_PER_TILE // CHUNK):
            pltpu.sync_copy(buf_v.at[2],
                            acc_s.at[pl.ds(row0 + t * CHUNK, CHUNK)])
        idx_cp.wait()
        plsc.subcore_barrier()

        def body(i, _):
            slot = lax.rem(i, 3)
            pltpu.make_async_copy(
                ea_hbm.at[pl.ds(base, CHUNK)], buf_v.at[slot], ld_sem
            ).wait()

            @pl.when(i > 0)
            def _():
                scatter_desc(i - 1, lax.rem(i - 1, 3)).wait()

            scatter_desc(i, slot).start(add=True)

            @pl.when(i + 2 < N_CHUNKS)
            def _():
                load(i + 2, lax.rem(i + 2, 3))
            return 0

        lax.fori_loop(0, N_CHUNKS, body, 0)
        scatter_desc(N_CHUNKS - 1, lax.rem(N_CHUNKS - 1, 3)).wait()
        plsc.subcore_barrier()

        for t in range(ROWS_PER_TILE // CHUNK):
            r = row0 + t * CHUNK
            pltpu.sync_copy(acc_s.at[pl.ds(r, CHUNK)], buf_v.at[0])
            pltpu.sync_copy(buf_v.at[0], out_hbm.at[cid, pl.ds(r, CHUNK)])

    return seg_sum(dst_idx, edge_attr)


def _bdot(a, b):
    return jnp.dot(a.astype(jnp.bfloat16), b.astype(jnp.bfloat16),
                   preferred_element_type=jnp.float32)


def _mlp_body(x_ref, a0_ref, a1_ref, w1_ref, b1_ref, w2_ref, b2_ref,
              w3_ref, b3_ref, o_ref):
    xb = x_ref[...]
    agg = a0_ref[0] + a1_ref[0]
    h = _bdot(xb, w1_ref[0:D_FEAT, :]) + _bdot(agg, w1_ref[D_FEAT:, :])
    h = h + b1_ref[...]
    h = h * jax.nn.sigmoid(h)
    h = _bdot(h, w2_ref[...]) + b2_ref[...]
    h = h * jax.nn.sigmoid(h)
    o_ref[...] = xb + _bdot(h, w3_ref[...]) + b3_ref[...]


def _tc_mlp(x, partials, W1, b1, W2, b2, W3, b3):
    BLK = 1000
    grid = (N_NODES // BLK,)
    rowspec = pl.BlockSpec((BLK, D_FEAT), lambda i: (i, 0))
    pspec = lambda c: pl.BlockSpec((1, BLK, D_FEAT), lambda i: (c, i, 0))
    whole = lambda shape: pl.BlockSpec(shape, lambda i: (0,) * len(shape))
    return pl.pallas_call(
        _mlp_body,
        grid=grid,
        in_specs=[
            rowspec, pspec(0), pspec(1),
            whole(W1.shape), whole(b1.shape),
            whole(W2.shape), whole(b2.shape),
            whole(W3.shape), whole(b3.shape),
        ],
        out_specs=rowspec,
        out_shape=jax.ShapeDtypeStruct((N_NODES, D_FEAT), jnp.float32),
    )(x, partials, partials, W1, b1, W2, b2, W3, b3)


def kernel(x, edge_index, edge_attr, W1, b1, W2, b2, W3, b3):
    dst_idx = edge_index.reshape(2, NW, N_CHUNKS, CHUNK)
    partials = _sc_segment_sum(dst_idx, edge_attr)
    return _tc_mlp(x, partials,
                   W1, b1.reshape(1, -1), W2, b2.reshape(1, -1),
                   W3, b3.reshape(1, -1))


# speedup vs baseline: 1.0988x; 1.0193x over previous
import functools

import jax
import jax.numpy as jnp
from jax import lax
from jax.experimental import pallas as pl
from jax.experimental.pallas import tpu as pltpu
from jax.experimental.pallas import tpu_sc as plsc

N_NODES = 10000
N_EDGES = 320000
D_FEAT = 128

NC = 2
NS = 16
NW = NC * NS
EDGES_PER_TILE = N_EDGES // NW
CHUNK = 80
N_CHUNKS = EDGES_PER_TILE // CHUNK
SUPER = 5
SUPER_ROWS = SUPER * CHUNK
N_SUPER = EDGES_PER_TILE // SUPER_ROWS
N_PAD = 10240
ROWS_PER_TILE = N_PAD // NS
DRAIN = 128
N_DRAIN = ROWS_PER_TILE // DRAIN


def _sc_segment_sum(dst_idx, edge_attr):
    mesh = plsc.VectorSubcoreMesh(core_axis_name="c", subcore_axis_name="s")

    @functools.partial(
        pl.kernel,
        mesh=mesh,
        out_type=jax.ShapeDtypeStruct((NC, N_PAD, D_FEAT), jnp.float32),
        scratch_types=[
            pltpu.VMEM((3, CHUNK, D_FEAT), jnp.float32),
            pltpu.VMEM((N_CHUNKS, CHUNK), jnp.int32),
            pltpu.VMEM_SHARED((N_PAD, D_FEAT), jnp.float32),
            pltpu.SemaphoreType.DMA,
            pltpu.SemaphoreType.DMA,
        ],
    )
    def seg_sum(idx_hbm, ea_hbm, out_hbm, buf_v, idx_v, acc_s,
                ld_sem, sc_sem):
        cid = lax.axis_index("c")
        sid = lax.axis_index("s")
        wid = sid * NC + cid
        base = wid * EDGES_PER_TILE
        idx_hbm = idx_hbm.at[1]

        def load(i, slot):
            return pltpu.async_copy(
                ea_hbm.at[pl.ds(base + i * CHUNK, CHUNK)], buf_v.at[slot],
                ld_sem)

        def scatter_desc(i, slot):
            return pltpu.make_async_copy(
                buf_v.at[slot], acc_s.at[idx_v.at[i]], sc_sem)

        idx_cp = pltpu.async_copy(idx_hbm.at[wid], idx_v, sc_sem)
        load(0, 0)
        load(1, 1)

        zval = jnp.zeros((16,), jnp.float32)

        def zrow(i, _):
            for j in range(D_FEAT // 16):
                buf_v[2, i, pl.ds(j * 16, 16)] = zval
            return 0

        lax.fori_loop(0, CHUNK, zrow, 0)
        row0 = sid * ROWS_PER_TILE
        for t in range(ROWS_PER_TILE // CHUNK):
            pltpu.sync_copy(buf_v.at[2],
                            acc_s.at[pl.ds(row0 + t * CHUNK, CHUNK)])
        idx_cp.wait()
        plsc.subcore_barrier()

        def body(i, _):
            slot = lax.rem(i, 3)
            pltpu.make_async_copy(
                ea_hbm.at[pl.ds(base, CHUNK)], buf_v.at[slot], ld_sem
            ).wait()

            @pl.when(i > 0)
            def _():
                scatter_desc(i - 1, lax.rem(i - 1, 3)).wait()

            scatter_desc(i, slot).start(add=True)

            @pl.when(i + 2 < N_CHUNKS)
            def _():
                load(i + 2, lax.rem(i + 2, 3))
            return 0

        lax.fori_loop(0, N_CHUNKS, body, 0)
        scatter_desc(N_CHUNKS - 1, lax.rem(N_CHUNKS - 1, 3)).wait()
        plsc.subcore_barrier()

        for t in range(ROWS_PER_TILE // CHUNK):
            r = row0 + t * CHUNK
            pltpu.sync_copy(acc_s.at[pl.ds(r, CHUNK)], buf_v.at[0])
            pltpu.sync_copy(buf_v.at[0], out_hbm.at[cid, pl.ds(r, CHUNK)])

    return seg_sum(dst_idx, edge_attr)


def _bdot(a, b):
    return jnp.dot(a.astype(jnp.bfloat16), b.astype(jnp.bfloat16),
                   preferred_element_type=jnp.float32)


def _mlp_body(x_ref, a0_ref, a1_ref, w1_ref, b1_ref, w2_ref, b2_ref,
              w3_ref, b3_ref, o_ref):
    xb = x_ref[...]
    agg = a0_ref[0] + a1_ref[0]
    h = _bdot(xb, w1_ref[0:D_FEAT, :]) + _bdot(agg, w1_ref[D_FEAT:, :])
    h = h + b1_ref[...]
    h = h * jax.nn.sigmoid(h)
    h = _bdot(h, w2_ref[...]) + b2_ref[...]
    h = h * jax.nn.sigmoid(h)
    o_ref[...] = xb + _bdot(h, w3_ref[...]) + b3_ref[...]


def _tc_mlp(x, partials, W1, b1, W2, b2, W3, b3):
    BLK = 2000
    grid = (N_NODES // BLK,)
    rowspec = pl.BlockSpec((BLK, D_FEAT), lambda i: (i, 0))
    pspec = lambda c: pl.BlockSpec((1, BLK, D_FEAT), lambda i: (c, i, 0))
    whole = lambda shape: pl.BlockSpec(shape, lambda i: (0,) * len(shape))
    return pl.pallas_call(
        _mlp_body,
        grid=grid,
        in_specs=[
            rowspec, pspec(0), pspec(1),
            whole(W1.shape), whole(b1.shape),
            whole(W2.shape), whole(b2.shape),
            whole(W3.shape), whole(b3.shape),
        ],
        out_specs=rowspec,
        out_shape=jax.ShapeDtypeStruct((N_NODES, D_FEAT), jnp.float32),
    )(x, partials, partials, W1, b1, W2, b2, W3, b3)


def kernel(x, edge_index, edge_attr, W1, b1, W2, b2, W3, b3):
    dst_idx = edge_index.reshape(2, NW, N_CHUNKS, CHUNK)
    partials = _sc_segment_sum(dst_idx, edge_attr)
    return _tc_mlp(x, partials,
                   W1, b1.reshape(1, -1), W2, b2.reshape(1, -1),
                   W3, b3.reshape(1, -1))
